# Initial kernel scaffold; baseline (speedup 1.0000x reference)
import functools

import jax
import jax.numpy as jnp
from jax import lax
from jax.experimental import pallas as pl
from jax.experimental.pallas import tpu as pltpu
from jax.experimental.pallas import tpu_sc as plsc

NC = 2
NS = 16
NW = NC * NS
L = 16

D_MODEL = 64
SCALE = 8.0

CHUNK = 128
NBUF = 4


def _make_emb_kernel(n_idx: int):
    assert n_idx % (NW * CHUNK) == 0
    per_w = n_idx // NW
    nch = per_w // CHUNK
    assert nch % NBUF == 0
    ngrp = nch // NBUF

    mesh = plsc.VectorSubcoreMesh(
        core_axis_name="c", subcore_axis_name="s",
        num_cores=NC, num_subcores=NS,
    )

    scratch = [pltpu.VMEM((nch, CHUNK), jnp.int32)]
    scratch += [pltpu.VMEM((CHUNK, D_MODEL), jnp.float32) for _ in range(NBUF)]
    scratch += [pltpu.SemaphoreType.DMA for _ in range(2 * NBUF)]

    @functools.partial(
        pl.kernel,
        out_type=jax.ShapeDtypeStruct((n_idx, D_MODEL), jnp.float32),
        mesh=mesh,
        scratch_types=scratch,
    )
    def emb(idx_hbm, table_hbm, out_hbm, idx_v, *rest):
        rows = rest[:NBUF]
        sem_in = rest[NBUF:2 * NBUF]
        sem_out = rest[2 * NBUF:]

        wid = lax.axis_index("s") * NC + lax.axis_index("c")
        base = wid * per_w

        pltpu.sync_copy(idx_hbm.at[wid], idx_v)

        for b in range(NBUF):
            pltpu.async_copy(table_hbm.at[idx_v.at[b]], rows[b], sem_in[b])

        def group(i, carry):
            for b in range(NBUF):
                g = i * NBUF + b
                r = rows[b]
                pltpu.make_async_copy(
                    table_hbm.at[pl.ds(0, CHUNK)], r, sem_in[b]).wait()

                def scale_row(j, c, r=r):
                    for k in range(D_MODEL // L):
                        r[j, pl.ds(k * L, L)] = r[j, pl.ds(k * L, L)] * SCALE
                    return c
                lax.fori_loop(0, CHUNK, scale_row, 0, unroll=4)

                pltpu.async_copy(
                    r, out_hbm.at[pl.ds(base + g * CHUNK, CHUNK)], sem_out[b])
                pltpu.make_async_copy(
                    r, out_hbm.at[pl.ds(0, CHUNK)], sem_out[b]).wait()

                @pl.when(i + 1 < ngrp)
                def _fire_next(b=b, g=g):
                    pltpu.async_copy(
                        table_hbm.at[idx_v.at[g + NBUF]], rows[b], sem_in[b])
            return carry

        lax.fori_loop(0, ngrp, group, 0)

    return emb


@functools.lru_cache(maxsize=None)
def _get_emb(n_idx: int):
    return _make_emb_kernel(n_idx)


def kernel(x, table):
    n_rows, n_cols = x.shape
    n_idx = n_rows * n_cols
    idx = x.astype(jnp.int32).reshape(NW, n_idx // (NW * CHUNK), CHUNK)
    out = _get_emb(n_idx)(idx, table)
    return out.reshape(n_rows, n_cols, D_MODEL)


# speedup vs baseline: 4.2509x; 4.2509x over previous
import functools

import jax
import jax.numpy as jnp
from jax import lax
from jax.experimental import pallas as pl
from jax.experimental.pallas import tpu as pltpu
from jax.experimental.pallas import tpu_sc as plsc

NC = 2
NS = 16
NW = NC * NS
L = 16

D_MODEL = 64
SCALE = 8.0

CHUNK = 128
NBUF = 4


def _make_emb_kernel(n_idx: int):
    assert n_idx % (NW * CHUNK) == 0
    per_w = n_idx // NW
    nch = per_w // CHUNK
    assert nch % NBUF == 0
    ngrp = nch // NBUF

    mesh = plsc.VectorSubcoreMesh(
        core_axis_name="c", subcore_axis_name="s",
        num_cores=NC, num_subcores=NS,
    )

    scratch = [pltpu.VMEM((nch, CHUNK), jnp.int32)]
    scratch += [pltpu.VMEM((CHUNK, D_MODEL), jnp.float32) for _ in range(NBUF)]
    scratch += [pltpu.SemaphoreType.DMA for _ in range(2 * NBUF)]

    @functools.partial(
        pl.kernel,
        out_type=jax.ShapeDtypeStruct((n_idx, D_MODEL), jnp.float32),
        mesh=mesh,
        scratch_types=scratch,
        compiler_params=pltpu.CompilerParams(use_tc_tiling_on_sc=False),
    )
    def emb(idx_hbm, table_hbm, out_hbm, idx_v, *rest):
        rows = rest[:NBUF]
        sem_in = rest[NBUF:2 * NBUF]
        sem_out = rest[2 * NBUF:]

        wid = lax.axis_index("s") * NC + lax.axis_index("c")
        base = wid * per_w

        pltpu.sync_copy(idx_hbm.at[wid], idx_v)

        for b in range(NBUF):
            pltpu.async_copy(table_hbm.at[idx_v.at[b]], rows[b], sem_in[b])

        def group(i, carry):
            for b in range(NBUF):
                g = i * NBUF + b
                r = rows[b]
                pltpu.make_async_copy(
                    table_hbm.at[pl.ds(0, CHUNK)], r, sem_in[b]).wait()

                def scale_row(j, c, r=r):
                    for k in range(D_MODEL // L):
                        r[j, pl.ds(k * L, L)] = r[j, pl.ds(k * L, L)] * SCALE
                    return c
                lax.fori_loop(0, CHUNK, scale_row, 0, unroll=4)

                pltpu.async_copy(
                    r, out_hbm.at[pl.ds(base + g * CHUNK, CHUNK)], sem_out[b])
                pltpu.make_async_copy(
                    r, out_hbm.at[pl.ds(0, CHUNK)], sem_out[b]).wait()

                @pl.when(i + 1 < ngrp)
                def _fire_next(b=b, g=g):
                    pltpu.async_copy(
                        table_hbm.at[idx_v.at[g + NBUF]], rows[b], sem_in[b])
            return carry

        lax.fori_loop(0, ngrp, group, 0)

    return emb


@functools.lru_cache(maxsize=None)
def _get_emb(n_idx: int):
    return _make_emb_kernel(n_idx)


def kernel(x, table):
    n_rows, n_cols = x.shape
    n_idx = n_rows * n_cols
    idx = x.astype(jnp.int32).reshape(NW, n_idx // (NW * CHUNK), CHUNK)
    out = _get_emb(n_idx)(idx, table)
    return out.reshape(n_rows, n_cols, D_MODEL)


# speedup vs baseline: 4.2646x; 1.0032x over previous
import functools

import jax
import jax.numpy as jnp
from jax import lax
from jax.experimental import pallas as pl
from jax.experimental.pallas import tpu as pltpu
from jax.experimental.pallas import tpu_sc as plsc

NC = 2
NS = 16
NW = NC * NS
L = 16

D_MODEL = 64
SCALE = 8.0

CHUNK = 128
NBUF = 8
AHEAD = 4


def _make_emb_kernel(n_idx: int):
    assert n_idx % (NW * CHUNK) == 0
    per_w = n_idx // NW
    nch = per_w // CHUNK
    assert nch % NBUF == 0
    ngrp = nch // NBUF

    mesh = plsc.VectorSubcoreMesh(
        core_axis_name="c", subcore_axis_name="s",
        num_cores=NC, num_subcores=NS,
    )

    scratch = [pltpu.VMEM((nch, CHUNK), jnp.int32)]
    scratch += [pltpu.VMEM((CHUNK, D_MODEL), jnp.float32) for _ in range(NBUF)]
    scratch += [pltpu.SemaphoreType.DMA for _ in range(2 * NBUF)]

    @functools.partial(
        pl.kernel,
        out_type=jax.ShapeDtypeStruct((n_idx, D_MODEL), jnp.float32),
        mesh=mesh,
        scratch_types=scratch,
        compiler_params=pltpu.CompilerParams(use_tc_tiling_on_sc=False),
    )
    def emb(idx_hbm, table_hbm, out_hbm, idx_v, *rest):
        rows = rest[:NBUF]
        sem_in = rest[NBUF:2 * NBUF]
        sem_out = rest[2 * NBUF:]

        wid = lax.axis_index("s") * NC + lax.axis_index("c")
        base = wid * per_w

        pltpu.sync_copy(idx_hbm.at[wid], idx_v)

        def fire_gather(f, bf):
            pltpu.async_copy(table_hbm.at[idx_v.at[f]], rows[bf], sem_in[bf])

        def drain_gather(bf):
            pltpu.make_async_copy(
                table_hbm.at[pl.ds(0, CHUNK)], rows[bf], sem_in[bf]).wait()

        def drain_store(bf):
            pltpu.make_async_copy(
                rows[bf], out_hbm.at[pl.ds(0, CHUNK)], sem_out[bf]).wait()

        for b in range(AHEAD):
            fire_gather(b, b)

        def group(i, carry):
            for b in range(NBUF):
                g = i * NBUF + b
                r = rows[b]
                drain_gather(b)

                def scale_row(j, c, r=r):
                    for k in range(D_MODEL // L):
                        r[j, pl.ds(k * L, L)] = r[j, pl.ds(k * L, L)] * SCALE
                    return c
                lax.fori_loop(0, CHUNK, scale_row, 0, unroll=4)

                pltpu.async_copy(
                    r, out_hbm.at[pl.ds(base + g * CHUNK, CHUNK)], sem_out[b])

                bf = (b + AHEAD) % NBUF
                if b + AHEAD < NBUF:
                    @pl.when(i > 0)
                    def _drain(bf=bf):
                        drain_store(bf)
                    fire_gather(g + AHEAD, bf)
                else:
                    drain_store(bf)

                    @pl.when(i + 1 < ngrp)
                    def _fire(g=g, bf=bf):
                        fire_gather(g + AHEAD, bf)
            return carry

        lax.fori_loop(0, ngrp, group, 0)

        for c in range(nch - (NBUF - AHEAD), nch):
            drain_store(c % NBUF)

    return emb


@functools.lru_cache(maxsize=None)
def _get_emb(n_idx: int):
    return _make_emb_kernel(n_idx)


def kernel(x, table):
    n_rows, n_cols = x.shape
    n_idx = n_rows * n_cols
    idx = x.astype(jnp.int32).reshape(NW, n_idx // (NW * CHUNK), CHUNK)
    out = _get_emb(n_idx)(idx, table)
    return out.reshape(n_rows, n_cols, D_MODEL)
